# baseline (speedup 1.0000x reference)
import jax
import jax.numpy as jnp
from jax import lax
from jax.experimental import pallas as pl
from jax.experimental.pallas import tpu as pltpu
from jax.experimental.pallas import tpu_sc as plsc

B = 16384
D = 16
NC = 2
NS = 16
NW = NC * NS
BPW = B // NW
G = BPW // D


def _sc_body(users_hbm, items_hbm, ue_hbm, ie_hbm, ub_hbm, ib_hbm, out_hbm,
             idx_u, idx_i, rows_u, rows_i, bias_u, bias_i, out_v, sem):
    wid = lax.axis_index("s") * NC + lax.axis_index("c")
    base = wid * BPW

    pltpu.sync_copy(users_hbm.at[pl.ds(base, BPW)], idx_u)
    pltpu.sync_copy(items_hbm.at[pl.ds(base, BPW)], idx_i)

    cu = pltpu.async_copy(ue_hbm.at[idx_u], rows_u, sem)
    ci = pltpu.async_copy(ie_hbm.at[idx_i], rows_i, sem)
    cbu = pltpu.async_copy(ub_hbm.at[idx_u], bias_u, sem)
    cbi = pltpu.async_copy(ib_hbm.at[idx_i], bias_i, sem)
    cu.wait()
    ci.wait()
    cbu.wait()
    cbi.wait()

    lane = lax.iota(jnp.int32, D)

    @pl.loop(0, G)
    def _(g):
        r0 = g * D
        acc = bias_u[pl.ds(r0, D)] + bias_i[pl.ds(r0, D)]
        for j in range(D):
            p = rows_u[r0 + j] * rows_i[r0 + j]
            s = jnp.sum(p)
            acc = acc + jnp.where(lane == j, s, jnp.float32(0.0))
        out_v[pl.ds(r0, D)] = acc

    pltpu.sync_copy(out_v, out_hbm.at[pl.ds(base, BPW)])


def kernel(users, items, user_embeddings, item_embeddings, user_biases, item_biases):
    users = users.astype(jnp.int32)
    items = items.astype(jnp.int32)
    ub = user_biases.reshape(-1)
    ib = item_biases.reshape(-1)

    cp = pltpu.CompilerParams(needs_layout_passes=False, use_tc_tiling_on_sc=False)
    mesh = plsc.VectorSubcoreMesh(core_axis_name="c", subcore_axis_name="s")
    run = pl.kernel(
        _sc_body,
        out_type=jax.ShapeDtypeStruct((B,), jnp.float32),
        mesh=mesh,
        scratch_types=[
            pltpu.VMEM((BPW,), jnp.int32),
            pltpu.VMEM((BPW,), jnp.int32),
            pltpu.VMEM((BPW, D), jnp.float32),
            pltpu.VMEM((BPW, D), jnp.float32),
            pltpu.VMEM((BPW,), jnp.float32),
            pltpu.VMEM((BPW,), jnp.float32),
            pltpu.VMEM((BPW,), jnp.float32),
            pltpu.SemaphoreType.DMA,
        ],
        compiler_params=cp,
    )
    return run(users, items, user_embeddings, item_embeddings, ub, ib)
